# Initial kernel scaffold; baseline (speedup 1.0000x reference)
import jax
import jax.numpy as jnp
from jax.experimental import pallas as pl


def kernel(x, edge_index, batch, W1, b1, W2, b2, W3, b3, g1, beta1, g2, beta2, W0, b0):
    raise NotImplementedError("write your pallas kernel here")


# speedup vs baseline: 8.6097x; 8.6097x over previous
import functools
import math

import jax
import jax.numpy as jnp
from jax import lax
from jax.experimental import pallas as pl
from jax.experimental.pallas import tpu as pltpu
from jax.experimental.pallas import tpu_sc as plsc

N = 10000
E = 160000
F_IN = 256
H = 256
HH = 128
C = 16
G = 64
BN_EPS = 1e-5
NPAD = 10240
EPAD = 163840
CH = 128
BR = 512
_BNS = 1.0 / math.sqrt(1.0 + BN_EPS)


def _sc_mesh():
    return plsc.VectorSubcoreMesh(
        core_axis_name="c", subcore_axis_name="s", num_cores=2, num_subcores=16)


_DW = 16
_DCH = EPAD // 32 // CH


def _deg_body(dst_hbm, deg_out, idx2d, ones2d, zb2d, acc_sh):
    c = lax.axis_index("c")
    s = lax.axis_index("s")
    w = c * 16 + s
    iota = lax.iota(jnp.int32, 16)
    e0 = jnp.where(iota == 0, 1.0, 0.0)
    z16 = jnp.zeros((16,), jnp.float32)
    for r in range(CH):
        ones2d[r, pl.ds(0, _DW)] = e0
        zb2d[r, pl.ds(0, _DW)] = z16
    sl = NPAD // 16
    for t in range(sl // CH):
        pltpu.sync_copy(zb2d, acc_sh.at[pl.ds(s * sl + t * CH, CH)])
    plsc.subcore_barrier()
    pltpu.sync_copy(dst_hbm.at[pl.ds(w * _DCH, _DCH)], idx2d)
    for j in range(_DCH):
        pltpu.sync_copy(ones2d, acc_sh.at[idx2d.at[j]], add=True)
    plsc.subcore_barrier()
    pltpu.sync_copy(acc_sh.at[pl.ds(s * sl, sl)],
                    deg_out.at[c, pl.ds(s * sl, sl)])


def _deg(dst2d):
    k = pl.kernel(
        _deg_body,
        out_type=jax.ShapeDtypeStruct((2, NPAD, _DW), jnp.float32),
        mesh=_sc_mesh(),
        scratch_types=[
            pltpu.VMEM((_DCH, CH), jnp.int32),
            pltpu.VMEM((CH, _DW), jnp.float32),
            pltpu.VMEM((CH, _DW), jnp.float32),
            pltpu.VMEM_SHARED((NPAD, _DW), jnp.float32),
        ],
    )
    return k(dst2d)


_EC = EPAD // 16
_NCH = _EC // CH


def _agg_body(src_hbm, dst_hbm, tab_hbm, out_hbm,
              src0, dst0, rows0, zbuf, acc, sem0):
    c = lax.axis_index("c")
    s = lax.axis_index("s")
    zz = jnp.zeros((16,), jnp.float32)
    for r in range(CH):
        for j in range(HH // 16):
            zbuf[r, pl.ds(j * 16, 16)] = zz
    sl = NPAD // 16
    for t in range(sl // CH):
        pltpu.sync_copy(zbuf, acc.at[pl.ds(s * sl + t * CH, CH)])
    plsc.subcore_barrier()

    base = s * _EC
    coff = c * NPAD

    def chunk(i, _):
        off = base + i * CH
        pltpu.sync_copy(src_hbm.at[pl.ds(off, CH)], src0)
        pltpu.sync_copy(dst_hbm.at[pl.ds(off, CH)], dst0)
        for jj in range(CH // 16):
            src0[pl.ds(jj * 16, 16)] = src0[pl.ds(jj * 16, 16)] + coff
        pltpu.async_copy(tab_hbm.at[src0], rows0, sem0).wait()
        pltpu.sync_copy(rows0, acc.at[dst0], add=True)
        return 0
    lax.fori_loop(0, _NCH, chunk, 0)

    plsc.subcore_barrier()
    for t in range(sl // CH):
        r0 = s * sl + t * CH
        pltpu.sync_copy(acc.at[pl.ds(r0, CH)], out_hbm.at[c, pl.ds(r0, CH)])


def _agg(srcp, dstp, table):
    k = pl.kernel(
        _agg_body,
        out_type=jax.ShapeDtypeStruct((2, NPAD, HH), jnp.float32),
        mesh=_sc_mesh(),
        scratch_types=[
            pltpu.VMEM((CH,), jnp.int32),
            pltpu.VMEM((CH,), jnp.int32),
            pltpu.VMEM((CH, HH), jnp.float32),
            pltpu.VMEM((CH, HH), jnp.float32),
            pltpu.VMEM_SHARED((NPAD, HH), jnp.float32),
            pltpu.SemaphoreType.DMA,
        ],
    )
    return k(srcp, dstp, table)


_RB = 16


def _segmax_body(z_hbm, batch_hbm, out_hbm, batch_v, buf, accv):
    c = lax.axis_index("c")
    s = lax.axis_index("s")
    w = c * 16 + s
    pltpu.sync_copy(batch_hbm, batch_v)
    lo = w * 2

    zi = jnp.zeros((16,), jnp.int32)

    def cbody(i, carry):
        a0, a1, a2 = carry
        v = batch_v[pl.ds(i * 16, 16)]
        a0 = a0 + jnp.where(v < lo, 1, 0)
        a1 = a1 + jnp.where(v < lo + 1, 1, 0)
        a2 = a2 + jnp.where(v < lo + 2, 1, 0)
        return (a0, a1, a2)
    a0, a1, a2 = lax.fori_loop(0, N // 16, cbody, (zi, zi, zi))
    c0 = a0[0]
    c1 = a1[0]
    c2 = a2[0]
    for j in range(1, 16):
        c0 = c0 + a0[j]
        c1 = c1 + a1[j]
        c2 = c2 + a2[j]

    ninf = jnp.full((16,), -jnp.inf, jnp.float32)
    starts = (c0, c1)
    ends = (c1, c2)
    for k2 in range(2):
        st = starts[k2]
        en = ends[k2]
        st8 = st - lax.rem(st, 8)
        for q in range(H // 16):
            accv[pl.ds(q * 16, 16)] = ninf
        nch = lax.div(en - st8 + (_RB - 1), _RB)

        def mbody(i, _):
            row0 = pl.multiple_of(st8 + i * _RB, 8)
            pltpu.sync_copy(z_hbm.at[pl.ds(row0, _RB)], buf)
            for j in range(_RB):
                rj = row0 + j
                @pl.when(jnp.logical_and(rj >= st, rj < en))
                def _():
                    for q in range(H // 16):
                        accv[pl.ds(q * 16, 16)] = jnp.maximum(
                            accv[pl.ds(q * 16, 16)], buf[j, pl.ds(q * 16, 16)])
            return 0
        lax.fori_loop(0, nch, mbody, 0)
        pltpu.sync_copy(accv, out_hbm.at[pl.ds((lo + k2) * H, H)])


def _segmax(z3, batch):
    k = pl.kernel(
        _segmax_body,
        out_type=jax.ShapeDtypeStruct((G * H,), jnp.float32),
        mesh=_sc_mesh(),
        scratch_types=[
            pltpu.VMEM((N,), jnp.int32),
            pltpu.VMEM((_RB, H), jnp.float32),
            pltpu.VMEM((H,), jnp.float32),
        ],
    )
    return k(z3, batch)


def _mm1_body(x_ref, w_ref, deg_ref, out_ref, dinv_ref):
    deg = deg_ref[0, :, 0] + deg_ref[1, :, 0] + 1.0
    dv = lax.rsqrt(deg)[:, None]
    h = jnp.dot(x_ref[...], w_ref[...], preferred_element_type=jnp.float32)
    hp = h * dv
    out_ref[0] = hp[:, :HH]
    out_ref[1] = hp[:, HH:]
    dinv_ref[...] = dv


def _mm1(x_pad, W1, deg2):
    return pl.pallas_call(
        _mm1_body,
        grid=(NPAD // BR,),
        in_specs=[
            pl.BlockSpec((BR, F_IN), lambda i: (i, 0)),
            pl.BlockSpec((F_IN, H), lambda i: (0, 0)),
            pl.BlockSpec((2, BR, _DW), lambda i: (0, i, 0)),
        ],
        out_specs=[
            pl.BlockSpec((2, BR, HH), lambda i: (0, i, 0)),
            pl.BlockSpec((BR, 1), lambda i: (i, 0)),
        ],
        out_shape=[
            jax.ShapeDtypeStruct((2, NPAD, HH), jnp.float32),
            jax.ShapeDtypeStruct((NPAD, 1), jnp.float32),
        ],
    )(x_pad, W1, deg2)


def _mmmid_body(agg_ref, hp_ref, dinv_ref, b_ref, g_ref, bt_ref, w_ref,
                out_ref):
    agg = jnp.concatenate([agg_ref[0], agg_ref[1]], axis=1)
    hp = jnp.concatenate([hp_ref[0], hp_ref[1]], axis=1)
    dv = dinv_ref[...]
    z = dv * (agg + hp) + b_ref[...]
    z = jnp.maximum(z * (g_ref[...] * _BNS) + bt_ref[...], 0.0)
    h = jnp.dot(z, w_ref[...], preferred_element_type=jnp.float32)
    hp2 = h * dv
    out_ref[0] = hp2[:, :HH]
    out_ref[1] = hp2[:, HH:]


def _mmmid(agg, hp, dinv, b, g, bt, W):
    return pl.pallas_call(
        _mmmid_body,
        grid=(NPAD // BR,),
        in_specs=[
            pl.BlockSpec((2, BR, HH), lambda i: (0, i, 0)),
            pl.BlockSpec((2, BR, HH), lambda i: (0, i, 0)),
            pl.BlockSpec((BR, 1), lambda i: (i, 0)),
            pl.BlockSpec((1, H), lambda i: (0, 0)),
            pl.BlockSpec((1, H), lambda i: (0, 0)),
            pl.BlockSpec((1, H), lambda i: (0, 0)),
            pl.BlockSpec((H, H), lambda i: (0, 0)),
        ],
        out_specs=pl.BlockSpec((2, BR, HH), lambda i: (0, i, 0)),
        out_shape=jax.ShapeDtypeStruct((2, NPAD, HH), jnp.float32),
    )(agg, hp, dinv, b, g, bt, W)


def _zfin_body(agg_ref, hp_ref, dinv_ref, b_ref, out_ref):
    agg = jnp.concatenate([agg_ref[0], agg_ref[1]], axis=1)
    hp = jnp.concatenate([hp_ref[0], hp_ref[1]], axis=1)
    out_ref[...] = dinv_ref[...] * (agg + hp) + b_ref[...]


def _zfin(agg, hp, dinv, b):
    return pl.pallas_call(
        _zfin_body,
        grid=(NPAD // BR,),
        in_specs=[
            pl.BlockSpec((2, BR, HH), lambda i: (0, i, 0)),
            pl.BlockSpec((2, BR, HH), lambda i: (0, i, 0)),
            pl.BlockSpec((BR, 1), lambda i: (i, 0)),
            pl.BlockSpec((1, H), lambda i: (0, 0)),
        ],
        out_specs=pl.BlockSpec((BR, H), lambda i: (i, 0)),
        out_shape=jax.ShapeDtypeStruct((NPAD, H), jnp.float32),
    )(agg, hp, dinv, b)


def _head_body(sm_ref, w_ref, b_ref, out_ref):
    o = jnp.dot(sm_ref[...], w_ref[...], preferred_element_type=jnp.float32)
    o = o + b_ref[...]
    m = jnp.max(o, axis=1, keepdims=True)
    e = o - m
    lse = jnp.log(jnp.sum(jnp.exp(e), axis=1, keepdims=True))
    out_ref[...] = e - lse


def _head(sm, W0, b0):
    return pl.pallas_call(
        _head_body,
        out_shape=jax.ShapeDtypeStruct((G, C), jnp.float32),
    )(sm, W0, b0)


def kernel(x, edge_index, batch, W1, b1, W2, b2, W3, b3,
           g1, beta1, g2, beta2, W0, b0):
    x_pad = jnp.concatenate(
        [x, jnp.zeros((NPAD - N, F_IN), jnp.float32)], axis=0)
    padn = EPAD - E
    pad_idx = N + (jnp.arange(padn, dtype=jnp.int32) % (NPAD - N))
    srcp = jnp.concatenate([edge_index[0], pad_idx])
    dstp = jnp.concatenate([edge_index[1], pad_idx])

    deg2 = _deg(dstp.reshape(EPAD // CH, CH))
    hp1, dinv = _mm1(x_pad, W1, deg2)
    agg1 = _agg(srcp, dstp, hp1.reshape(2 * NPAD, HH))
    hp2 = _mmmid(agg1, hp1, dinv, b1.reshape(1, H), g1.reshape(1, H),
                 beta1.reshape(1, H), W2)
    agg2 = _agg(srcp, dstp, hp2.reshape(2 * NPAD, HH))
    hp3 = _mmmid(agg2, hp2, dinv, b2.reshape(1, H), g2.reshape(1, H),
                 beta2.reshape(1, H), W3)
    agg3 = _agg(srcp, dstp, hp3.reshape(2 * NPAD, HH))
    z3 = _zfin(agg3, hp3, dinv, b3.reshape(1, H))
    sm = _segmax(z3, batch).reshape(G, H)
    return _head(sm, W0, b0.reshape(1, C))


# speedup vs baseline: 16.0168x; 1.8603x over previous
import functools
import math

import jax
import jax.numpy as jnp
from jax import lax
from jax.experimental import pallas as pl
from jax.experimental.pallas import tpu as pltpu
from jax.experimental.pallas import tpu_sc as plsc

N = 10000
E = 160000
F_IN = 256
H = 256
HH = 128
C = 16
G = 64
BN_EPS = 1e-5
NPAD = 10240
EPAD = 163840
CH = 128
BR = 512
_BNS = 1.0 / math.sqrt(1.0 + BN_EPS)


def _sc_mesh():
    return plsc.VectorSubcoreMesh(
        core_axis_name="c", subcore_axis_name="s", num_cores=2, num_subcores=16)


_DW = 16
_DCH = EPAD // 32 // CH


def _deg_body(dst_hbm, deg_out, idx2d, ones2d, zb2d, acc_sh):
    c = lax.axis_index("c")
    s = lax.axis_index("s")
    w = c * 16 + s
    iota = lax.iota(jnp.int32, 16)
    e0 = jnp.where(iota == 0, 1.0, 0.0)
    z16 = jnp.zeros((16,), jnp.float32)
    for r in range(CH):
        ones2d[r, pl.ds(0, _DW)] = e0
        zb2d[r, pl.ds(0, _DW)] = z16
    sl = NPAD // 16
    for t in range(sl // CH):
        pltpu.sync_copy(zb2d, acc_sh.at[pl.ds(s * sl + t * CH, CH)])
    plsc.subcore_barrier()
    pltpu.sync_copy(dst_hbm.at[pl.ds(w * _DCH, _DCH)], idx2d)
    for j in range(_DCH):
        pltpu.sync_copy(ones2d, acc_sh.at[idx2d.at[j]], add=True)
    plsc.subcore_barrier()
    pltpu.sync_copy(acc_sh.at[pl.ds(s * sl, sl)],
                    deg_out.at[c, pl.ds(s * sl, sl)])


def _deg(dst2d):
    k = pl.kernel(
        _deg_body,
        out_type=jax.ShapeDtypeStruct((2, NPAD, _DW), jnp.float32),
        mesh=_sc_mesh(),
        scratch_types=[
            pltpu.VMEM((_DCH, CH), jnp.int32),
            pltpu.VMEM((CH, _DW), jnp.float32),
            pltpu.VMEM((CH, _DW), jnp.float32),
            pltpu.VMEM_SHARED((NPAD, _DW), jnp.float32),
        ],
    )
    return k(dst2d)


_EC = EPAD // 16
_NCH = _EC // CH


def _agg_body(src_hbm, dst2d, tab_hbm, out_hbm,
              dstv, sb0, sb1, rows0, rows1, acc,
              g0, g1, i0, i1):
    c = lax.axis_index("c")
    s = lax.axis_index("s")
    zz = jnp.zeros((16,), jnp.float32)
    for r in range(CH):
        for j in range(HH // 16):
            rows0[r, pl.ds(j * 16, 16)] = zz
    sl = NPAD // 16
    for t in range(sl // CH):
        pltpu.sync_copy(rows0, acc.at[pl.ds(s * sl + t * CH, CH)])
    plsc.subcore_barrier()

    pltpu.sync_copy(dst2d.at[pl.ds(s * _NCH, _NCH)], dstv)
    base = s * _EC
    coff = c * NPAD
    rows = (rows0, rows1)
    sbs = (sb0, sb1)
    gsems = (g0, g1)
    isems = (i0, i1)

    def iload(ic, b, pred):
        @pl.when(pred)
        def _():
            pltpu.async_copy(
                src_hbm.at[pl.ds(base + ic * CH, CH)], sbs[b], isems[b])

    def gissue(b):
        pltpu.make_async_copy(
            src_hbm.at[pl.ds(0, CH)], sbs[b], isems[b]).wait()
        for j in range(CH // 16):
            sbs[b][pl.ds(j * 16, 16)] = sbs[b][pl.ds(j * 16, 16)] + coff
        pltpu.async_copy(tab_hbm.at[sbs[b]], rows[b], gsems[b])

    iload(0, 0, True)
    iload(1, 1, True)
    gissue(0)
    gissue(1)

    def outer(o, _):
        for b in range(2):
            i = o * 2 + b
            pltpu.make_async_copy(
                tab_hbm.at[pl.ds(0, CH)], rows[b], gsems[b]).wait()
            iload(jnp.minimum(i + 2, _NCH - 1), b, i + 2 < _NCH)
            pltpu.sync_copy(rows[b], acc.at[dstv.at[i]], add=True)

            @pl.when(i + 2 < _NCH)
            def _():
                gissue(b)
        return 0
    lax.fori_loop(0, _NCH // 2, outer, 0)

    plsc.subcore_barrier()
    for t in range(sl // CH):
        r0 = s * sl + t * CH
        pltpu.sync_copy(acc.at[pl.ds(r0, CH)], out_hbm.at[c, pl.ds(r0, CH)])


def _agg(srcp, dst2d, table):
    k = pl.kernel(
        _agg_body,
        out_type=jax.ShapeDtypeStruct((2, NPAD, HH), jnp.float32),
        mesh=_sc_mesh(),
        scratch_types=[
            pltpu.VMEM((_NCH, CH), jnp.int32),
            pltpu.VMEM((CH,), jnp.int32),
            pltpu.VMEM((CH,), jnp.int32),
            pltpu.VMEM((CH, HH), jnp.float32),
            pltpu.VMEM((CH, HH), jnp.float32),
            pltpu.VMEM_SHARED((NPAD, HH), jnp.float32),
            pltpu.SemaphoreType.DMA,
            pltpu.SemaphoreType.DMA,
            pltpu.SemaphoreType.DMA,
            pltpu.SemaphoreType.DMA,
        ],
    )
    return k(srcp, dst2d, table)


_RB = 16


def _segmax_body(z_hbm, batch_hbm, out_hbm, batch_v, buf, accv):
    c = lax.axis_index("c")
    s = lax.axis_index("s")
    w = c * 16 + s
    pltpu.sync_copy(batch_hbm, batch_v)
    lo = w * 2

    zi = jnp.zeros((16,), jnp.int32)

    def cbody(i, carry):
        a0, a1, a2 = carry
        v = batch_v[pl.ds(i * 16, 16)]
        a0 = a0 + jnp.where(v < lo, 1, 0)
        a1 = a1 + jnp.where(v < lo + 1, 1, 0)
        a2 = a2 + jnp.where(v < lo + 2, 1, 0)
        return (a0, a1, a2)
    a0, a1, a2 = lax.fori_loop(0, N // 16, cbody, (zi, zi, zi))
    c0 = a0[0]
    c1 = a1[0]
    c2 = a2[0]
    for j in range(1, 16):
        c0 = c0 + a0[j]
        c1 = c1 + a1[j]
        c2 = c2 + a2[j]

    ninf = jnp.full((16,), -jnp.inf, jnp.float32)
    starts = (c0, c1)
    ends = (c1, c2)
    for k2 in range(2):
        st = starts[k2]
        en = ends[k2]
        st8 = st - lax.rem(st, 8)
        for q in range(H // 16):
            accv[pl.ds(q * 16, 16)] = ninf
        nch = lax.div(en - st8 + (_RB - 1), _RB)

        def mbody(i, _):
            row0 = pl.multiple_of(st8 + i * _RB, 8)
            pltpu.sync_copy(z_hbm.at[pl.ds(row0, _RB)], buf)
            for j in range(_RB):
                rj = row0 + j
                @pl.when(jnp.logical_and(rj >= st, rj < en))
                def _():
                    for q in range(H // 16):
                        accv[pl.ds(q * 16, 16)] = jnp.maximum(
                            accv[pl.ds(q * 16, 16)], buf[j, pl.ds(q * 16, 16)])
            return 0
        lax.fori_loop(0, nch, mbody, 0)
        pltpu.sync_copy(accv, out_hbm.at[pl.ds((lo + k2) * H, H)])


def _segmax(z3, batch):
    k = pl.kernel(
        _segmax_body,
        out_type=jax.ShapeDtypeStruct((G * H,), jnp.float32),
        mesh=_sc_mesh(),
        scratch_types=[
            pltpu.VMEM((N,), jnp.int32),
            pltpu.VMEM((_RB, H), jnp.float32),
            pltpu.VMEM((H,), jnp.float32),
        ],
    )
    return k(z3, batch)


def _mm1_body(x_ref, w_ref, deg_ref, out_ref, dinv_ref):
    deg = deg_ref[0, :, 0] + deg_ref[1, :, 0] + 1.0
    dv = lax.rsqrt(deg)[:, None]
    h = jnp.dot(x_ref[...], w_ref[...], preferred_element_type=jnp.float32)
    hp = h * dv
    out_ref[0] = hp[:, :HH]
    out_ref[1] = hp[:, HH:]
    dinv_ref[...] = dv


def _mm1(x_pad, W1, deg2):
    return pl.pallas_call(
        _mm1_body,
        grid=(NPAD // BR,),
        in_specs=[
            pl.BlockSpec((BR, F_IN), lambda i: (i, 0)),
            pl.BlockSpec((F_IN, H), lambda i: (0, 0)),
            pl.BlockSpec((2, BR, _DW), lambda i: (0, i, 0)),
        ],
        out_specs=[
            pl.BlockSpec((2, BR, HH), lambda i: (0, i, 0)),
            pl.BlockSpec((BR, 1), lambda i: (i, 0)),
        ],
        out_shape=[
            jax.ShapeDtypeStruct((2, NPAD, HH), jnp.float32),
            jax.ShapeDtypeStruct((NPAD, 1), jnp.float32),
        ],
    )(x_pad, W1, deg2)


def _mmmid_body(agg_ref, hp_ref, dinv_ref, b_ref, g_ref, bt_ref, w_ref,
                out_ref):
    agg = jnp.concatenate([agg_ref[0], agg_ref[1]], axis=1)
    hp = jnp.concatenate([hp_ref[0], hp_ref[1]], axis=1)
    dv = dinv_ref[...]
    z = dv * (agg + hp) + b_ref[...]
    z = jnp.maximum(z * (g_ref[...] * _BNS) + bt_ref[...], 0.0)
    h = jnp.dot(z, w_ref[...], preferred_element_type=jnp.float32)
    hp2 = h * dv
    out_ref[0] = hp2[:, :HH]
    out_ref[1] = hp2[:, HH:]


def _mmmid(agg, hp, dinv, b, g, bt, W):
    return pl.pallas_call(
        _mmmid_body,
        grid=(NPAD // BR,),
        in_specs=[
            pl.BlockSpec((2, BR, HH), lambda i: (0, i, 0)),
            pl.BlockSpec((2, BR, HH), lambda i: (0, i, 0)),
            pl.BlockSpec((BR, 1), lambda i: (i, 0)),
            pl.BlockSpec((1, H), lambda i: (0, 0)),
            pl.BlockSpec((1, H), lambda i: (0, 0)),
            pl.BlockSpec((1, H), lambda i: (0, 0)),
            pl.BlockSpec((H, H), lambda i: (0, 0)),
        ],
        out_specs=pl.BlockSpec((2, BR, HH), lambda i: (0, i, 0)),
        out_shape=jax.ShapeDtypeStruct((2, NPAD, HH), jnp.float32),
    )(agg, hp, dinv, b, g, bt, W)


def _zfin_body(agg_ref, hp_ref, dinv_ref, b_ref, out_ref):
    agg = jnp.concatenate([agg_ref[0], agg_ref[1]], axis=1)
    hp = jnp.concatenate([hp_ref[0], hp_ref[1]], axis=1)
    out_ref[...] = dinv_ref[...] * (agg + hp) + b_ref[...]


def _zfin(agg, hp, dinv, b):
    return pl.pallas_call(
        _zfin_body,
        grid=(NPAD // BR,),
        in_specs=[
            pl.BlockSpec((2, BR, HH), lambda i: (0, i, 0)),
            pl.BlockSpec((2, BR, HH), lambda i: (0, i, 0)),
            pl.BlockSpec((BR, 1), lambda i: (i, 0)),
            pl.BlockSpec((1, H), lambda i: (0, 0)),
        ],
        out_specs=pl.BlockSpec((BR, H), lambda i: (i, 0)),
        out_shape=jax.ShapeDtypeStruct((NPAD, H), jnp.float32),
    )(agg, hp, dinv, b)


def _head_body(sm_ref, w_ref, b_ref, out_ref):
    o = jnp.dot(sm_ref[...], w_ref[...], preferred_element_type=jnp.float32)
    o = o + b_ref[...]
    m = jnp.max(o, axis=1, keepdims=True)
    e = o - m
    lse = jnp.log(jnp.sum(jnp.exp(e), axis=1, keepdims=True))
    out_ref[...] = e - lse


def _head(sm, W0, b0):
    return pl.pallas_call(
        _head_body,
        out_shape=jax.ShapeDtypeStruct((G, C), jnp.float32),
    )(sm, W0, b0)


def kernel(x, edge_index, batch, W1, b1, W2, b2, W3, b3,
           g1, beta1, g2, beta2, W0, b0):
    x_pad = jnp.concatenate(
        [x, jnp.zeros((NPAD - N, F_IN), jnp.float32)], axis=0)
    padn = EPAD - E
    pad_idx = N + (jnp.arange(padn, dtype=jnp.int32) % (NPAD - N))
    srcp = jnp.concatenate([edge_index[0], pad_idx])
    dstp = jnp.concatenate([edge_index[1], pad_idx])

    dst2d = dstp.reshape(EPAD // CH, CH)
    deg2 = _deg(dst2d)
    hp1, dinv = _mm1(x_pad, W1, deg2)
    agg1 = _agg(srcp, dst2d, hp1.reshape(2 * NPAD, HH))
    hp2 = _mmmid(agg1, hp1, dinv, b1.reshape(1, H), g1.reshape(1, H),
                 beta1.reshape(1, H), W2)
    agg2 = _agg(srcp, dst2d, hp2.reshape(2 * NPAD, HH))
    hp3 = _mmmid(agg2, hp2, dinv, b2.reshape(1, H), g2.reshape(1, H),
                 beta2.reshape(1, H), W3)
    agg3 = _agg(srcp, dst2d, hp3.reshape(2 * NPAD, HH))
    z3 = _zfin(agg3, hp3, dinv, b3.reshape(1, H))
    sm = _segmax(z3, batch).reshape(G, H)
    return _head(sm, W0, b0.reshape(1, C))
